# baseline (speedup 1.0000x reference)
import functools

import jax
import jax.numpy as jnp
import numpy as np
from jax import lax
from jax.experimental import pallas as pl
from jax.experimental.pallas import tpu as pltpu
from jax.experimental.pallas import tpu_sc as plsc

N = 10000
E = 320000
D = 128
H = 256
K = 16

NC = 2
NS = 16
NW = NC * NS
EPW = E // NW
CH = 40
NCHUNK = EPW // CH
NBUF = 5
ROFF = 624
RWIN = 640

_SELU_ALPHA = 1.6732632423543772
_SELU_SCALE = 1.0507009873554805
_TWO_M = float(E)

_sc_mesh = functools.partial(
    plsc.VectorSubcoreMesh, core_axis_name="c", subcore_axis_name="s",
    num_cores=NC, num_subcores=NS)


def _agg_pipeline(x_hbm, out_hbm, gidx_v, sidx_v, bufs, gsems, ssems, acc,
                  c, row0, nchunk):
  pltpu.sync_copy(x_hbm.at[pl.ds(row0, RWIN)], acc.at[pl.ds(row0, RWIN)])
  plsc.subcore_barrier()

  def start_g(j, b):
    pltpu.async_copy(x_hbm.at[gidx_v.at[j]], bufs.at[b], gsems.at[b])

  def wait_g(j, b):
    pltpu.make_async_copy(x_hbm.at[gidx_v.at[j]], bufs.at[b],
                          gsems.at[b]).wait()

  def start_s(j, b):
    pltpu.async_copy(bufs.at[b], acc.at[sidx_v.at[j]], ssems.at[b],
                     add=True)

  def wait_s(j, b):
    pltpu.make_async_copy(bufs.at[b], acc.at[sidx_v.at[j]],
                          ssems.at[b]).wait()

  start_g(0, 0)
  start_g(1, 1)
  start_g(2, 2)
  for b in range(NBUF):
    j = b
    if j >= 2:
      wait_s(j - 2, (b + 3) % NBUF)
    start_g(j + 3, (b + 3) % NBUF)
    wait_g(j, b)
    start_s(j, b)

  def body(j0, carry):
    for b in range(NBUF):
      j = j0 * NBUF + b
      wait_s(j - 2, (b + 3) % NBUF)
      start_g(j + 3, (b + 3) % NBUF)
      wait_g(j, b)
      start_s(j, b)
    return carry

  lax.fori_loop(1, nchunk // NBUF - 1, body, 0)

  for b in range(NBUF):
    j = nchunk - NBUF + b
    wait_s(j - 2, (b + 3) % NBUF)
    if j + 3 < nchunk:
      start_g(j + 3, (b + 3) % NBUF)
    wait_g(j, b)
    start_s(j, b)
  wait_s(nchunk - 2, (nchunk - 2) % NBUF)
  wait_s(nchunk - 1, (nchunk - 1) % NBUF)

  plsc.subcore_barrier()
  pltpu.sync_copy(acc.at[pl.ds(row0, RWIN)],
                  out_hbm.at[c, pl.ds(row0, RWIN)])


@functools.cache
def _make_spmm(W, ch, nchunk, nrows, gd, sd):
  @functools.partial(
      pl.kernel,
      out_type=jax.ShapeDtypeStruct((NC, nrows, W), jnp.float32),
      mesh=_sc_mesh(),
      compiler_params=pltpu.CompilerParams(use_tc_tiling_on_sc=False),
      scratch_types=[
          pltpu.VMEM((nchunk, ch), jnp.int32),
          pltpu.VMEM((nchunk, ch), jnp.int32),
          pltpu.VMEM((NBUF, ch, W), jnp.float32),
          pltpu.SemaphoreType.DMA((NBUF,)),
          pltpu.SemaphoreType.DMA((NBUF,)),
          pltpu.VMEM_SHARED((nrows, W), jnp.float32),
      ],
  )
  def spmm(x_hbm, eidx_hbm, out_hbm, gidx_v, sidx_v, bufs,
           gsems, ssems, acc):
    c = lax.axis_index("c")
    s = lax.axis_index("s")
    wid = s * NC + c
    row0 = pl.multiple_of(s * ROFF, 8)
    pltpu.sync_copy(eidx_hbm.at[gd, wid], gidx_v)
    pltpu.sync_copy(eidx_hbm.at[sd, wid], sidx_v)
    _agg_pipeline(x_hbm, out_hbm, gidx_v, sidx_v, bufs, gsems, ssems,
                  acc, c, row0, nchunk)

  return spmm


@functools.cache
def _make_spmm_dual():
  @functools.partial(
      pl.kernel,
      out_type=(jax.ShapeDtypeStruct((NC, N, D), jnp.float32),
                jax.ShapeDtypeStruct((NC, N, D), jnp.float32)),
      mesh=_sc_mesh(),
      compiler_params=pltpu.CompilerParams(use_tc_tiling_on_sc=False),
      scratch_types=[
          pltpu.VMEM((NCHUNK, CH), jnp.int32),
          pltpu.VMEM((NCHUNK, CH), jnp.int32),
          pltpu.VMEM((NBUF, CH, D), jnp.float32),
          pltpu.SemaphoreType.DMA((NBUF,)),
          pltpu.SemaphoreType.DMA((NBUF,)),
          pltpu.VMEM_SHARED((N, D), jnp.float32),
      ],
  )
  def spmm2(xa_hbm, xb_hbm, eidx_hbm, outa_hbm, outb_hbm,
            gidx_v, sidx_v, bufs, gsems, ssems, acc):
    c = lax.axis_index("c")
    s = lax.axis_index("s")
    wid = s * NC + c
    row0 = pl.multiple_of(s * ROFF, 8)
    pltpu.sync_copy(eidx_hbm.at[0, wid], gidx_v)
    pltpu.sync_copy(eidx_hbm.at[1, wid], sidx_v)
    _agg_pipeline(xa_hbm, outa_hbm, gidx_v, sidx_v, bufs, gsems, ssems,
                  acc, c, row0, NCHUNK)
    _agg_pipeline(xb_hbm, outb_hbm, gidx_v, sidx_v, bufs, gsems, ssems,
                  acc, c, row0, NCHUNK)

  return spmm2


@functools.cache
def _make_hist():
  @functools.partial(
      pl.kernel,
      out_type=jax.ShapeDtypeStruct((NC, 2, N, 16), jnp.float32),
      mesh=_sc_mesh(),
      compiler_params=pltpu.CompilerParams(use_tc_tiling_on_sc=False),
      scratch_types=[
          pltpu.VMEM((NCHUNK, CH), jnp.int32),
          pltpu.VMEM((NCHUNK, CH), jnp.int32),
          pltpu.VMEM((CH, 16), jnp.float32),
          pltpu.SemaphoreType.DMA,
          pltpu.SemaphoreType.DMA,
          pltpu.VMEM_SHARED((N, 16), jnp.float32),
          pltpu.VMEM_SHARED((N, 16), jnp.float32),
      ],
  )
  def hist(zeros_hbm, ones_hbm, eidx_hbm, out_hbm,
           src_v, dst_v, ones_v, dsem, ssem, accd, accs):
    c = lax.axis_index("c")
    s = lax.axis_index("s")
    wid = s * NC + c
    row0 = pl.multiple_of(s * ROFF, 8)
    pltpu.sync_copy(zeros_hbm.at[pl.ds(row0, RWIN)],
                    accd.at[pl.ds(row0, RWIN)])
    pltpu.sync_copy(zeros_hbm.at[pl.ds(row0, RWIN)],
                    accs.at[pl.ds(row0, RWIN)])
    pltpu.sync_copy(ones_hbm, ones_v)
    pltpu.sync_copy(eidx_hbm.at[0, wid], src_v)
    pltpu.sync_copy(eidx_hbm.at[1, wid], dst_v)
    plsc.subcore_barrier()

    def start_pair(j):
      pltpu.async_copy(ones_v, accd.at[dst_v.at[j]], dsem, add=True)
      pltpu.async_copy(ones_v, accs.at[src_v.at[j]], ssem, add=True)

    def drain_pair(j):
      pltpu.make_async_copy(ones_v, accd.at[dst_v.at[j]], dsem).wait()
      pltpu.make_async_copy(ones_v, accs.at[src_v.at[j]], ssem).wait()

    for j in range(4):
      start_pair(j)

    def body(j, carry):
      start_pair(j + 4)
      drain_pair(j)
      return carry

    lax.fori_loop(0, NCHUNK - 4, body, 0)
    for j in range(NCHUNK - 4, NCHUNK):
      drain_pair(j)
    plsc.subcore_barrier()
    pltpu.sync_copy(accd.at[pl.ds(row0, RWIN)],
                    out_hbm.at[c, 0, pl.ds(row0, RWIN)])
    pltpu.sync_copy(accs.at[pl.ds(row0, RWIN)],
                    out_hbm.at[c, 1, pl.ds(row0, RWIN)])

  return hist


def _hist_call(zeros16, ones16, e4h):
  return _make_hist()(zeros16, ones16, e4h)


def _spmm_call(W, ch, nchunk, nrows, gd, sd, x, e4):
  return _make_spmm(W, ch, nchunk, nrows, gd, sd)(x, e4)


def _spmm_dual_call(xa, xb, e4):
  return _make_spmm_dual()(xa, xb, e4)


_BLK = 2000
_NBLK = N // _BLK


def _selu(x):
  return _SELU_SCALE * jnp.where(
      x > 0, x, _SELU_ALPHA * (jnp.exp(x) - 1.0))


def _k1_body(x_ref, hist_ref, xs0_ref, dinvw_ref, sdegw_ref):
  h = hist_ref[...]
  deg = 1.0 + h[0, 0, :, 0] + h[1, 0, :, 0]
  dinv = lax.rsqrt(deg)
  sdeg = h[0, 1, :, 0] + h[1, 1, :, 0]
  xs0_ref[...] = x_ref[...] * dinv[:, None]
  dinvw_ref[...] = jnp.broadcast_to(dinv[:, None], (_BLK, D))
  sdegw_ref[...] = jnp.broadcast_to(sdeg[:, None], (_BLK, D))


def _k1(x, hist):
  return pl.pallas_call(
      _k1_body,
      grid=(_NBLK,),
      in_specs=[
          pl.BlockSpec((_BLK, D), lambda i: (i, 0)),
          pl.BlockSpec((NC, 2, _BLK, 16), lambda i: (0, 0, i, 0)),
      ],
      out_specs=[
          pl.BlockSpec((_BLK, D), lambda i: (i, 0)),
          pl.BlockSpec((_BLK, D), lambda i: (i, 0)),
          pl.BlockSpec((_BLK, D), lambda i: (i, 0)),
      ],
      out_shape=[
          jax.ShapeDtypeStruct((N, D), jnp.float32),
          jax.ShapeDtypeStruct((N, D), jnp.float32),
          jax.ShapeDtypeStruct((N, D), jnp.float32),
      ],
  )(x, hist)


def _k2_body(o1_ref, xs0_ref, dinvw_ref, w1_ref, b1_ref,
             xs1a_ref, xs1b_ref):
  dinvw = dinvw_ref[...]
  agg = (o1_ref[0] + o1_ref[1] - xs0_ref[...]) * dinvw
  h = jnp.dot(agg, w1_ref[...], preferred_element_type=jnp.float32)
  x1 = _selu(h + b1_ref[...])
  xs1 = x1 * dinvw[:, :1]
  xs1a_ref[...] = xs1[:, :D]
  xs1b_ref[...] = xs1[:, D:]


def _k2(o1, xs0, dinvw, W1, b1):
  return pl.pallas_call(
      _k2_body,
      grid=(_NBLK,),
      in_specs=[
          pl.BlockSpec((NC, _BLK, D), lambda i: (0, i, 0)),
          pl.BlockSpec((_BLK, D), lambda i: (i, 0)),
          pl.BlockSpec((_BLK, D), lambda i: (i, 0)),
          pl.BlockSpec((D, H), lambda i: (0, 0)),
          pl.BlockSpec((1, H), lambda i: (0, 0)),
      ],
      out_specs=[
          pl.BlockSpec((_BLK, D), lambda i: (i, 0)),
          pl.BlockSpec((_BLK, D), lambda i: (i, 0)),
      ],
      out_shape=[
          jax.ShapeDtypeStruct((N, D), jnp.float32),
          jax.ShapeDtypeStruct((N, D), jnp.float32),
      ],
  )(o1, xs0, dinvw, W1, b1)


def _k3_body(o2a_ref, o2b_ref, xs1a_ref, xs1b_ref, dinvw_ref,
             w2_ref, b2_ref, wa_ref, ba_ref, s_ref):
  dinvw = dinvw_ref[...]
  xs1a = xs1a_ref[...]
  xs1b = xs1b_ref[...]
  agga = (o2a_ref[0] + o2a_ref[1] - xs1a) * dinvw
  aggb = (o2b_ref[0] + o2b_ref[1] - xs1b) * dinvw
  agg = jnp.concatenate([agga, aggb], axis=1)
  x1 = jnp.concatenate([xs1a, xs1b], axis=1) / dinvw[:, :1]
  h = jnp.dot(agg, w2_ref[...], preferred_element_type=jnp.float32)
  x2 = _selu(h + b2_ref[...]) + x1
  logits = jnp.dot(x2, wa_ref[...], preferred_element_type=jnp.float32)
  logits = logits + ba_ref[...]
  mx = jnp.max(logits, axis=1, keepdims=True)
  ex = jnp.exp(logits - mx)
  s_ref[...] = ex / jnp.sum(ex, axis=1, keepdims=True)


def _k3(o2a, o2b, xs1a, xs1b, dinvw, W2, b2, Wa, ba):
  return pl.pallas_call(
      _k3_body,
      grid=(_NBLK,),
      in_specs=[
          pl.BlockSpec((NC, _BLK, D), lambda i: (0, i, 0)),
          pl.BlockSpec((NC, _BLK, D), lambda i: (0, i, 0)),
          pl.BlockSpec((_BLK, D), lambda i: (i, 0)),
          pl.BlockSpec((_BLK, D), lambda i: (i, 0)),
          pl.BlockSpec((_BLK, D), lambda i: (i, 0)),
          pl.BlockSpec((H, H), lambda i: (0, 0)),
          pl.BlockSpec((1, H), lambda i: (0, 0)),
          pl.BlockSpec((H, K), lambda i: (0, 0)),
          pl.BlockSpec((1, K), lambda i: (0, 0)),
      ],
      out_specs=pl.BlockSpec((_BLK, K), lambda i: (i, 0)),
      out_shape=jax.ShapeDtypeStruct((N, K), jnp.float32),
  )(o2a, o2b, xs1a, xs1b, dinvw, W2, b2, Wa, ba)


def _k4_body(oas_ref, s_ref, sdegw_ref, x_ref,
             pooled_ref, spec_ref, coll_ref, tot_ref, entl_ref,
             m_acc, v_acc):
  i = pl.program_id(0)

  @pl.when(i == 0)
  def _init():
    m_acc[...] = jnp.zeros((K, D), jnp.float32)
    v_acc[...] = jnp.zeros((8, K), jnp.float32)

  s_blk = s_ref[...]
  as_blk = oas_ref[0] + oas_ref[1] - 2.0 * s_blk
  u_part = jnp.sum(sdegw_ref[:, :K] * s_blk, axis=0)
  cs_part = jnp.sum(s_blk, axis=0)
  tr_part = jnp.sum(s_blk * as_blk, axis=0)
  ent_part = jnp.sum(s_blk * jnp.log(s_blk + 1e-08), axis=0)
  zeros4 = jnp.zeros((4, K), jnp.float32)
  upd = jnp.concatenate(
      [u_part[None], cs_part[None], tr_part[None], ent_part[None], zeros4],
      axis=0)
  v_acc[...] += upd
  m_acc[...] += lax.dot_general(
      s_blk, x_ref[...], (((0,), (0,)), ((), ())),
      preferred_element_type=jnp.float32)

  @pl.when(i == _NBLK - 1)
  def _fin():
    u = v_acc[0, :]
    cs = v_acc[1, :]
    tr = jnp.sum(v_acc[2, :])
    ent_sum = jnp.sum(v_acc[3, :])
    normalizer = jnp.sum(u * u) / _TWO_M
    spectral = -(tr - K * normalizer) / _TWO_M
    collapse = jnp.sqrt(jnp.sum(cs * cs)) / N * np.sqrt(float(K)) - 1.0
    entropy = -ent_sum / N
    ent_loss = -0.1 * entropy
    pooled_ref[...] = m_acc[...] / (cs + 1e-08)[:, None]
    spec_ref[...] = spectral[None, None]
    coll_ref[...] = collapse[None, None]
    tot_ref[...] = (spectral + collapse + ent_loss)[None, None]
    entl_ref[...] = ent_loss[None, None]


def _k4(oas, s, sdegw, x):
  return pl.pallas_call(
      _k4_body,
      grid=(_NBLK,),
      in_specs=[
          pl.BlockSpec((NC, _BLK, K), lambda i: (0, i, 0)),
          pl.BlockSpec((_BLK, K), lambda i: (i, 0)),
          pl.BlockSpec((_BLK, D), lambda i: (i, 0)),
          pl.BlockSpec((_BLK, D), lambda i: (i, 0)),
      ],
      out_specs=[
          pl.BlockSpec((K, D), lambda i: (0, 0)),
          pl.BlockSpec((1, 1), lambda i: (0, 0)),
          pl.BlockSpec((1, 1), lambda i: (0, 0)),
          pl.BlockSpec((1, 1), lambda i: (0, 0)),
          pl.BlockSpec((1, 1), lambda i: (0, 0)),
      ],
      out_shape=[
          jax.ShapeDtypeStruct((K, D), jnp.float32),
          jax.ShapeDtypeStruct((1, 1), jnp.float32),
          jax.ShapeDtypeStruct((1, 1), jnp.float32),
          jax.ShapeDtypeStruct((1, 1), jnp.float32),
          jax.ShapeDtypeStruct((1, 1), jnp.float32),
      ],
      scratch_shapes=[
          pltpu.VMEM((K, D), jnp.float32),
          pltpu.VMEM((8, K), jnp.float32),
      ],
  )(oas, s, sdegw, x)


def kernel(embeddings, edge_index, W1, b1, W2, b2, Wa, ba):
  e4 = edge_index.reshape(2, NW, NCHUNK, CH)

  zeros16 = jnp.zeros((N, 16), jnp.float32)
  ones16 = jnp.ones((CH, 16), jnp.float32)

  hist = _hist_call(zeros16, ones16, e4)
  xs0, dinvw, sdegw = _k1(embeddings, hist)
  o1 = _spmm_call(D, CH, NCHUNK, N, 0, 1, xs0, e4)
  xs1a, xs1b = _k2(o1, xs0, dinvw, W1, b1.reshape(1, H))
  o2a, o2b = _spmm_dual_call(xs1a, xs1b, e4)
  s = _k3(o2a, o2b, xs1a, xs1b, dinvw, W2, b2.reshape(1, H),
          Wa, ba.reshape(1, K))
  oas = _spmm_call(K, CH, NCHUNK, N, 1, 0, s, e4)
  pooled, spec, coll, tot, entl = _k4(oas, s, sdegw, embeddings)
  return (s, pooled, spec.reshape(()), coll.reshape(()),
          tot.reshape(()), entl.reshape(()))


# speedup vs baseline: 1.0557x; 1.0557x over previous
import functools

import jax
import jax.numpy as jnp
import numpy as np
from jax import lax
from jax.experimental import pallas as pl
from jax.experimental.pallas import tpu as pltpu
from jax.experimental.pallas import tpu_sc as plsc

N = 10000
E = 320000
D = 128
H = 256
K = 16

NC = 2
NS = 16
NW = NC * NS
EPW = E // NW
CH = 40
NCHUNK = EPW // CH
NBUF = 5
CHB = 1000
NCHB = EPW // CHB
ROFF = 624
RWIN = 640

_SELU_ALPHA = 1.6732632423543772
_SELU_SCALE = 1.0507009873554805
_TWO_M = float(E)

_sc_mesh = functools.partial(
    plsc.VectorSubcoreMesh, core_axis_name="c", subcore_axis_name="s",
    num_cores=NC, num_subcores=NS)


def _agg_pipeline(x_hbm, out_hbm, gidx_v, sidx_v, bufs, gsems, ssems, acc,
                  c, row0, nchunk):
  pltpu.sync_copy(x_hbm.at[pl.ds(row0, RWIN)], acc.at[pl.ds(row0, RWIN)])
  plsc.subcore_barrier()

  def start_g(j, b):
    pltpu.async_copy(x_hbm.at[gidx_v.at[j]], bufs.at[b], gsems.at[b])

  def wait_g(j, b):
    pltpu.make_async_copy(x_hbm.at[gidx_v.at[j]], bufs.at[b],
                          gsems.at[b]).wait()

  def start_s(j, b):
    pltpu.async_copy(bufs.at[b], acc.at[sidx_v.at[j]], ssems.at[b],
                     add=True)

  def wait_s(j, b):
    pltpu.make_async_copy(bufs.at[b], acc.at[sidx_v.at[j]],
                          ssems.at[b]).wait()

  start_g(0, 0)
  start_g(1, 1)
  start_g(2, 2)
  for b in range(NBUF):
    j = b
    if j >= 2:
      wait_s(j - 2, (b + 3) % NBUF)
    start_g(j + 3, (b + 3) % NBUF)
    wait_g(j, b)
    start_s(j, b)

  def body(j0, carry):
    for b in range(NBUF):
      j = j0 * NBUF + b
      wait_s(j - 2, (b + 3) % NBUF)
      start_g(j + 3, (b + 3) % NBUF)
      wait_g(j, b)
      start_s(j, b)
    return carry

  lax.fori_loop(1, nchunk // NBUF - 1, body, 0)

  for b in range(NBUF):
    j = nchunk - NBUF + b
    wait_s(j - 2, (b + 3) % NBUF)
    if j + 3 < nchunk:
      start_g(j + 3, (b + 3) % NBUF)
    wait_g(j, b)
    start_s(j, b)
  wait_s(nchunk - 2, (nchunk - 2) % NBUF)
  wait_s(nchunk - 1, (nchunk - 1) % NBUF)

  plsc.subcore_barrier()
  pltpu.sync_copy(acc.at[pl.ds(row0, RWIN)],
                  out_hbm.at[c, pl.ds(row0, RWIN)])


@functools.cache
def _make_spmm(W, ch, nchunk, nrows, gd, sd):
  @functools.partial(
      pl.kernel,
      out_type=jax.ShapeDtypeStruct((NC, nrows, W), jnp.float32),
      mesh=_sc_mesh(),
      compiler_params=pltpu.CompilerParams(use_tc_tiling_on_sc=False),
      scratch_types=[
          pltpu.VMEM((nchunk, ch), jnp.int32),
          pltpu.VMEM((nchunk, ch), jnp.int32),
          pltpu.VMEM((NBUF, ch, W), jnp.float32),
          pltpu.SemaphoreType.DMA((NBUF,)),
          pltpu.SemaphoreType.DMA((NBUF,)),
          pltpu.VMEM_SHARED((nrows, W), jnp.float32),
      ],
  )
  def spmm(x_hbm, eidx_hbm, out_hbm, gidx_v, sidx_v, bufs,
           gsems, ssems, acc):
    c = lax.axis_index("c")
    s = lax.axis_index("s")
    wid = s * NC + c
    row0 = pl.multiple_of(s * ROFF, 8)
    pltpu.sync_copy(eidx_hbm.at[gd, wid], gidx_v)
    pltpu.sync_copy(eidx_hbm.at[sd, wid], sidx_v)
    _agg_pipeline(x_hbm, out_hbm, gidx_v, sidx_v, bufs, gsems, ssems,
                  acc, c, row0, nchunk)

  return spmm


@functools.cache
def _make_spmm_dual():
  @functools.partial(
      pl.kernel,
      out_type=(jax.ShapeDtypeStruct((NC, N, D), jnp.float32),
                jax.ShapeDtypeStruct((NC, N, D), jnp.float32)),
      mesh=_sc_mesh(),
      compiler_params=pltpu.CompilerParams(use_tc_tiling_on_sc=False),
      scratch_types=[
          pltpu.VMEM((NCHUNK, CH), jnp.int32),
          pltpu.VMEM((NCHUNK, CH), jnp.int32),
          pltpu.VMEM((NBUF, CH, D), jnp.float32),
          pltpu.SemaphoreType.DMA((NBUF,)),
          pltpu.SemaphoreType.DMA((NBUF,)),
          pltpu.VMEM_SHARED((N, D), jnp.float32),
      ],
  )
  def spmm2(xa_hbm, xb_hbm, eidx_hbm, outa_hbm, outb_hbm,
            gidx_v, sidx_v, bufs, gsems, ssems, acc):
    c = lax.axis_index("c")
    s = lax.axis_index("s")
    wid = s * NC + c
    row0 = pl.multiple_of(s * ROFF, 8)
    pltpu.sync_copy(eidx_hbm.at[0, wid], gidx_v)
    pltpu.sync_copy(eidx_hbm.at[1, wid], sidx_v)
    _agg_pipeline(xa_hbm, outa_hbm, gidx_v, sidx_v, bufs, gsems, ssems,
                  acc, c, row0, NCHUNK)
    _agg_pipeline(xb_hbm, outb_hbm, gidx_v, sidx_v, bufs, gsems, ssems,
                  acc, c, row0, NCHUNK)

  return spmm2


@functools.cache
def _make_hist():
  @functools.partial(
      pl.kernel,
      out_type=jax.ShapeDtypeStruct((NC, 2, N, 16), jnp.float32),
      mesh=_sc_mesh(),
      compiler_params=pltpu.CompilerParams(use_tc_tiling_on_sc=False),
      scratch_types=[
          pltpu.VMEM((NCHB, CHB), jnp.int32),
          pltpu.VMEM((NCHB, CHB), jnp.int32),
          pltpu.VMEM((CHB, 16), jnp.float32),
          pltpu.SemaphoreType.DMA,
          pltpu.SemaphoreType.DMA,
          pltpu.VMEM_SHARED((N, 16), jnp.float32),
          pltpu.VMEM_SHARED((N, 16), jnp.float32),
      ],
  )
  def hist(zeros_hbm, ones_hbm, eidx_hbm, out_hbm,
           src_v, dst_v, ones_v, dsem, ssem, accd, accs):
    c = lax.axis_index("c")
    s = lax.axis_index("s")
    wid = s * NC + c
    row0 = pl.multiple_of(s * ROFF, 8)
    pltpu.sync_copy(zeros_hbm.at[pl.ds(row0, RWIN)],
                    accd.at[pl.ds(row0, RWIN)])
    pltpu.sync_copy(zeros_hbm.at[pl.ds(row0, RWIN)],
                    accs.at[pl.ds(row0, RWIN)])
    pltpu.sync_copy(ones_hbm, ones_v)
    pltpu.sync_copy(eidx_hbm.at[0, wid], src_v)
    pltpu.sync_copy(eidx_hbm.at[1, wid], dst_v)
    plsc.subcore_barrier()

    def start_pair(j):
      pltpu.async_copy(ones_v, accd.at[dst_v.at[j]], dsem, add=True)
      pltpu.async_copy(ones_v, accs.at[src_v.at[j]], ssem, add=True)

    def drain_pair(j):
      pltpu.make_async_copy(ones_v, accd.at[dst_v.at[j]], dsem).wait()
      pltpu.make_async_copy(ones_v, accs.at[src_v.at[j]], ssem).wait()

    for j in range(4):
      start_pair(j)

    def body(j, carry):
      start_pair(j + 4)
      drain_pair(j)
      return carry

    lax.fori_loop(0, NCHB - 4, body, 0)
    for j in range(NCHB - 4, NCHB):
      drain_pair(j)
    plsc.subcore_barrier()
    pltpu.sync_copy(accd.at[pl.ds(row0, RWIN)],
                    out_hbm.at[c, 0, pl.ds(row0, RWIN)])
    pltpu.sync_copy(accs.at[pl.ds(row0, RWIN)],
                    out_hbm.at[c, 1, pl.ds(row0, RWIN)])

  return hist


def _hist_call(zeros16, ones16, e4h):
  return _make_hist()(zeros16, ones16, e4h)


def _spmm_call(W, ch, nchunk, nrows, gd, sd, x, e4):
  return _make_spmm(W, ch, nchunk, nrows, gd, sd)(x, e4)


def _spmm_dual_call(xa, xb, e4):
  return _make_spmm_dual()(xa, xb, e4)


_BLK = 2000
_NBLK = N // _BLK


def _selu(x):
  return _SELU_SCALE * jnp.where(
      x > 0, x, _SELU_ALPHA * (jnp.exp(x) - 1.0))


def _k1_body(x_ref, hist_ref, xs0_ref, dinvw_ref, sdegw_ref):
  h = hist_ref[...]
  deg = 1.0 + h[0, 0, :, 0] + h[1, 0, :, 0]
  dinv = lax.rsqrt(deg)
  sdeg = h[0, 1, :, 0] + h[1, 1, :, 0]
  xs0_ref[...] = x_ref[...] * dinv[:, None]
  dinvw_ref[...] = jnp.broadcast_to(dinv[:, None], (_BLK, D))
  sdegw_ref[...] = jnp.broadcast_to(sdeg[:, None], (_BLK, D))


def _k1(x, hist):
  return pl.pallas_call(
      _k1_body,
      grid=(_NBLK,),
      in_specs=[
          pl.BlockSpec((_BLK, D), lambda i: (i, 0)),
          pl.BlockSpec((NC, 2, _BLK, 16), lambda i: (0, 0, i, 0)),
      ],
      out_specs=[
          pl.BlockSpec((_BLK, D), lambda i: (i, 0)),
          pl.BlockSpec((_BLK, D), lambda i: (i, 0)),
          pl.BlockSpec((_BLK, D), lambda i: (i, 0)),
      ],
      out_shape=[
          jax.ShapeDtypeStruct((N, D), jnp.float32),
          jax.ShapeDtypeStruct((N, D), jnp.float32),
          jax.ShapeDtypeStruct((N, D), jnp.float32),
      ],
  )(x, hist)


def _k2_body(o1_ref, xs0_ref, dinvw_ref, w1_ref, b1_ref,
             xs1a_ref, xs1b_ref):
  dinvw = dinvw_ref[...]
  agg = (o1_ref[0] + o1_ref[1] - xs0_ref[...]) * dinvw
  h = jnp.dot(agg, w1_ref[...], preferred_element_type=jnp.float32)
  x1 = _selu(h + b1_ref[...])
  xs1 = x1 * dinvw[:, :1]
  xs1a_ref[...] = xs1[:, :D]
  xs1b_ref[...] = xs1[:, D:]


def _k2(o1, xs0, dinvw, W1, b1):
  return pl.pallas_call(
      _k2_body,
      grid=(_NBLK,),
      in_specs=[
          pl.BlockSpec((NC, _BLK, D), lambda i: (0, i, 0)),
          pl.BlockSpec((_BLK, D), lambda i: (i, 0)),
          pl.BlockSpec((_BLK, D), lambda i: (i, 0)),
          pl.BlockSpec((D, H), lambda i: (0, 0)),
          pl.BlockSpec((1, H), lambda i: (0, 0)),
      ],
      out_specs=[
          pl.BlockSpec((_BLK, D), lambda i: (i, 0)),
          pl.BlockSpec((_BLK, D), lambda i: (i, 0)),
      ],
      out_shape=[
          jax.ShapeDtypeStruct((N, D), jnp.float32),
          jax.ShapeDtypeStruct((N, D), jnp.float32),
      ],
  )(o1, xs0, dinvw, W1, b1)


def _k3_body(o2a_ref, o2b_ref, xs1a_ref, xs1b_ref, dinvw_ref,
             w2_ref, b2_ref, wa_ref, ba_ref, s_ref):
  dinvw = dinvw_ref[...]
  xs1a = xs1a_ref[...]
  xs1b = xs1b_ref[...]
  agga = (o2a_ref[0] + o2a_ref[1] - xs1a) * dinvw
  aggb = (o2b_ref[0] + o2b_ref[1] - xs1b) * dinvw
  agg = jnp.concatenate([agga, aggb], axis=1)
  x1 = jnp.concatenate([xs1a, xs1b], axis=1) / dinvw[:, :1]
  h = jnp.dot(agg, w2_ref[...], preferred_element_type=jnp.float32)
  x2 = _selu(h + b2_ref[...]) + x1
  logits = jnp.dot(x2, wa_ref[...], preferred_element_type=jnp.float32)
  logits = logits + ba_ref[...]
  mx = jnp.max(logits, axis=1, keepdims=True)
  ex = jnp.exp(logits - mx)
  s_ref[...] = ex / jnp.sum(ex, axis=1, keepdims=True)


def _k3(o2a, o2b, xs1a, xs1b, dinvw, W2, b2, Wa, ba):
  return pl.pallas_call(
      _k3_body,
      grid=(_NBLK,),
      in_specs=[
          pl.BlockSpec((NC, _BLK, D), lambda i: (0, i, 0)),
          pl.BlockSpec((NC, _BLK, D), lambda i: (0, i, 0)),
          pl.BlockSpec((_BLK, D), lambda i: (i, 0)),
          pl.BlockSpec((_BLK, D), lambda i: (i, 0)),
          pl.BlockSpec((_BLK, D), lambda i: (i, 0)),
          pl.BlockSpec((H, H), lambda i: (0, 0)),
          pl.BlockSpec((1, H), lambda i: (0, 0)),
          pl.BlockSpec((H, K), lambda i: (0, 0)),
          pl.BlockSpec((1, K), lambda i: (0, 0)),
      ],
      out_specs=pl.BlockSpec((_BLK, K), lambda i: (i, 0)),
      out_shape=jax.ShapeDtypeStruct((N, K), jnp.float32),
  )(o2a, o2b, xs1a, xs1b, dinvw, W2, b2, Wa, ba)


def _k4_body(oas_ref, s_ref, sdegw_ref, x_ref,
             pooled_ref, spec_ref, coll_ref, tot_ref, entl_ref,
             m_acc, v_acc):
  i = pl.program_id(0)

  @pl.when(i == 0)
  def _init():
    m_acc[...] = jnp.zeros((K, D), jnp.float32)
    v_acc[...] = jnp.zeros((8, K), jnp.float32)

  s_blk = s_ref[...]
  as_blk = oas_ref[0] + oas_ref[1] - 2.0 * s_blk
  u_part = jnp.sum(sdegw_ref[:, :K] * s_blk, axis=0)
  cs_part = jnp.sum(s_blk, axis=0)
  tr_part = jnp.sum(s_blk * as_blk, axis=0)
  ent_part = jnp.sum(s_blk * jnp.log(s_blk + 1e-08), axis=0)
  zeros4 = jnp.zeros((4, K), jnp.float32)
  upd = jnp.concatenate(
      [u_part[None], cs_part[None], tr_part[None], ent_part[None], zeros4],
      axis=0)
  v_acc[...] += upd
  m_acc[...] += lax.dot_general(
      s_blk, x_ref[...], (((0,), (0,)), ((), ())),
      preferred_element_type=jnp.float32)

  @pl.when(i == _NBLK - 1)
  def _fin():
    u = v_acc[0, :]
    cs = v_acc[1, :]
    tr = jnp.sum(v_acc[2, :])
    ent_sum = jnp.sum(v_acc[3, :])
    normalizer = jnp.sum(u * u) / _TWO_M
    spectral = -(tr - K * normalizer) / _TWO_M
    collapse = jnp.sqrt(jnp.sum(cs * cs)) / N * np.sqrt(float(K)) - 1.0
    entropy = -ent_sum / N
    ent_loss = -0.1 * entropy
    pooled_ref[...] = m_acc[...] / (cs + 1e-08)[:, None]
    spec_ref[...] = spectral[None, None]
    coll_ref[...] = collapse[None, None]
    tot_ref[...] = (spectral + collapse + ent_loss)[None, None]
    entl_ref[...] = ent_loss[None, None]


def _k4(oas, s, sdegw, x):
  return pl.pallas_call(
      _k4_body,
      grid=(_NBLK,),
      in_specs=[
          pl.BlockSpec((NC, _BLK, K), lambda i: (0, i, 0)),
          pl.BlockSpec((_BLK, K), lambda i: (i, 0)),
          pl.BlockSpec((_BLK, D), lambda i: (i, 0)),
          pl.BlockSpec((_BLK, D), lambda i: (i, 0)),
      ],
      out_specs=[
          pl.BlockSpec((K, D), lambda i: (0, 0)),
          pl.BlockSpec((1, 1), lambda i: (0, 0)),
          pl.BlockSpec((1, 1), lambda i: (0, 0)),
          pl.BlockSpec((1, 1), lambda i: (0, 0)),
          pl.BlockSpec((1, 1), lambda i: (0, 0)),
      ],
      out_shape=[
          jax.ShapeDtypeStruct((K, D), jnp.float32),
          jax.ShapeDtypeStruct((1, 1), jnp.float32),
          jax.ShapeDtypeStruct((1, 1), jnp.float32),
          jax.ShapeDtypeStruct((1, 1), jnp.float32),
          jax.ShapeDtypeStruct((1, 1), jnp.float32),
      ],
      scratch_shapes=[
          pltpu.VMEM((K, D), jnp.float32),
          pltpu.VMEM((8, K), jnp.float32),
      ],
  )(oas, s, sdegw, x)


def kernel(embeddings, edge_index, W1, b1, W2, b2, Wa, ba):
  e4 = edge_index.reshape(2, NW, NCHUNK, CH)
  e4b = edge_index.reshape(2, NW, NCHB, CHB)

  zeros16 = jnp.zeros((N, 16), jnp.float32)
  ones16 = jnp.ones((CHB, 16), jnp.float32)

  hist = _hist_call(zeros16, ones16, e4b)
  xs0, dinvw, sdegw = _k1(embeddings, hist)
  o1 = _spmm_call(D, CH, NCHUNK, N, 0, 1, xs0, e4)
  xs1a, xs1b = _k2(o1, xs0, dinvw, W1, b1.reshape(1, H))
  o2a, o2b = _spmm_dual_call(xs1a, xs1b, e4)
  s = _k3(o2a, o2b, xs1a, xs1b, dinvw, W2, b2.reshape(1, H),
          Wa, ba.reshape(1, K))
  oas = _spmm_call(K, CHB, NCHB, N, 1, 0, s, e4b)
  pooled, spec, coll, tot, entl = _k4(oas, s, sdegw, embeddings)
  return (s, pooled, spec.reshape(()), coll.reshape(()),
          tot.reshape(()), entl.reshape(()))
